# baseline (speedup 1.0000x reference)
import functools

import jax
import jax.numpy as jnp
from jax.experimental import pallas as pl
from jax.experimental.pallas import tpu as pltpu

NUM_VARS = 128
K = 5
TT = 1024


def _conv_body(xc_ref, xh_ref, w_ref, b_ref, out_hbm, xs_ref, sem):
    i = pl.program_id(0)
    j = pl.program_id(1)
    nt = pl.num_programs(1)
    lin = i * nt + j
    total = pl.num_programs(0) * nt
    slot = jax.lax.rem(lin, 2)

    def _wait_for(lin2):
        i2 = jax.lax.div(lin2, nt)
        j2 = jax.lax.rem(lin2, nt)
        slot2 = jax.lax.rem(lin2, 2)

        @pl.when(j2 == 0)
        def _():
            pltpu.make_async_copy(
                xs_ref.at[slot2, 1:TT],
                out_hbm.at[i2, :TT - 1, 0, :],
                sem.at[slot2]).wait()

        @pl.when(j2 != 0)
        def _():
            pltpu.make_async_copy(
                xs_ref.at[slot2, 0:TT],
                out_hbm.at[i2, pl.ds(j2 * TT - 1, TT), 0, :],
                sem.at[slot2]).wait()

    @pl.when(lin >= 2)
    def _():
        _wait_for(lin - 2)

    halo8 = jnp.where(j == 0, 0.0, xh_ref[0]).astype(jnp.bfloat16)
    xcb = xc_ref[0].astype(jnp.bfloat16)

    s0 = jnp.concatenate([halo8[4:8], xcb[:TT - 4]], axis=0)
    s2 = jnp.concatenate([halo8[6:8], xcb[:TT - 2]], axis=0)

    def dot(a, wk):
        return jax.lax.dot_general(
            a, wk, dimension_numbers=(((1,), (1,)), ((), ())),
            preferred_element_type=jnp.float32)

    w0, w1, w2, w3, w4 = (w_ref[k].astype(jnp.bfloat16) for k in range(K))
    row = jax.lax.broadcasted_iota(jnp.int32, (NUM_VARS, NUM_VARS), 0)
    col = jax.lax.broadcasted_iota(jnp.int32, (NUM_VARS, NUM_VARS), 1)
    w4 = jnp.where(row == col, 0.0, w4)

    odd = dot(s2, w1) + dot(xcb, w3)
    hrow = dot(halo8, w1)[5:6] + dot(halo8, w3)[7:8]
    odd_sh = jnp.concatenate([hrow, odd[:TT - 1]], axis=0)

    acc = (jnp.broadcast_to(b_ref[0][None, :], (TT, NUM_VARS)).astype(jnp.float32)
           + dot(s0, w0) + dot(s2, w2) + dot(xcb, w4) + odd_sh)
    xs_ref[slot] = acc

    @pl.when(j == 0)
    def _():
        pltpu.make_async_copy(
            xs_ref.at[slot, 1:TT],
            out_hbm.at[i, :TT - 1, 0, :],
            sem.at[slot]).start()

    @pl.when(j != 0)
    def _():
        pltpu.make_async_copy(
            xs_ref.at[slot, 0:TT],
            out_hbm.at[i, pl.ds(j * TT - 1, TT), 0, :],
            sem.at[slot]).start()

    @pl.when(lin == total - 1)
    def _():
        _wait_for(lin - 1)
        _wait_for(lin)


@functools.partial(jax.jit, static_argnames=())
def kernel(spikes, weight, bias):
    b, t, n, _ = spikes.shape
    nt = t // TT
    ttb = TT // 8
    x = jnp.reshape(spikes, (b, t, n))
    w = jnp.transpose(weight, (2, 0, 1))
    bias2 = bias[None, :]
    out = pl.pallas_call(
        _conv_body,
        grid=(b, nt),
        in_specs=[
            pl.BlockSpec((1, TT, n), lambda i, j: (i, j, 0)),
            pl.BlockSpec((1, 8, n),
                         lambda i, j: (i, jnp.maximum(j * ttb - 1, 0), 0)),
            pl.BlockSpec((K, n, n), lambda i, j: (0, 0, 0)),
            pl.BlockSpec((1, n), lambda i, j: (0, 0)),
        ],
        out_specs=pl.BlockSpec(memory_space=pl.ANY),
        out_shape=jax.ShapeDtypeStruct((b, t - 1, 1, n), jnp.float32),
        scratch_shapes=[
            pltpu.MemorySpace.VMEM((2, TT, n), jnp.float32),
            pltpu.SemaphoreType.DMA((2,)),
        ],
    )(x, x, w, bias2)
    return jnp.reshape(out, (b, t - 1, n, 1))


# speedup vs baseline: 1.6926x; 1.6926x over previous
import functools

import jax
import jax.numpy as jnp
from jax.experimental import pallas as pl
from jax.experimental.pallas import tpu as pltpu

NUM_VARS = 128
K = 5
TT = 4096


def _conv_body(xc_ref, xh_ref, w_ref, b_ref, out_hbm, xs_ref, sem):
    i = pl.program_id(0)
    j = pl.program_id(1)
    nt = pl.num_programs(1)
    lin = i * nt + j
    total = pl.num_programs(0) * nt
    slot = jax.lax.rem(lin, 2)

    def _wait_for(lin2):
        i2 = jax.lax.div(lin2, nt)
        j2 = jax.lax.rem(lin2, nt)
        slot2 = jax.lax.rem(lin2, 2)

        @pl.when(j2 == 0)
        def _():
            pltpu.make_async_copy(
                xs_ref.at[slot2, 1:TT],
                out_hbm.at[i2, :TT - 1, 0, :],
                sem.at[slot2]).wait()

        @pl.when(j2 != 0)
        def _():
            pltpu.make_async_copy(
                xs_ref.at[slot2, 0:TT],
                out_hbm.at[i2, pl.ds(j2 * TT - 1, TT), 0, :],
                sem.at[slot2]).wait()

    @pl.when(lin >= 2)
    def _():
        _wait_for(lin - 2)

    halo8 = jnp.where(j == 0, 0.0, xh_ref[0]).astype(jnp.bfloat16)
    xcb = xc_ref[0].astype(jnp.bfloat16)

    s0 = jnp.concatenate([halo8[4:8], xcb[:TT - 4]], axis=0)
    s2 = jnp.concatenate([halo8[6:8], xcb[:TT - 2]], axis=0)

    def dot(a, wk):
        return jax.lax.dot_general(
            a, wk, dimension_numbers=(((1,), (1,)), ((), ())),
            preferred_element_type=jnp.float32)

    w0, w1, w2, w3, w4 = (w_ref[k].astype(jnp.bfloat16) for k in range(K))
    row = jax.lax.broadcasted_iota(jnp.int32, (NUM_VARS, NUM_VARS), 0)
    col = jax.lax.broadcasted_iota(jnp.int32, (NUM_VARS, NUM_VARS), 1)
    w4 = jnp.where(row == col, 0.0, w4)

    odd = dot(s2, w1) + dot(xcb, w3)
    hrow = dot(halo8, w1)[5:6] + dot(halo8, w3)[7:8]
    odd_sh = jnp.concatenate([hrow, odd[:TT - 1]], axis=0)

    acc = (jnp.broadcast_to(b_ref[0][None, :], (TT, NUM_VARS)).astype(jnp.float32)
           + dot(s0, w0) + dot(s2, w2) + dot(xcb, w4) + odd_sh)
    xs_ref[slot] = acc

    @pl.when(j == 0)
    def _():
        pltpu.make_async_copy(
            xs_ref.at[slot, 1:TT],
            out_hbm.at[i, :TT - 1, 0, :],
            sem.at[slot]).start()

    @pl.when(j != 0)
    def _():
        pltpu.make_async_copy(
            xs_ref.at[slot, 0:TT],
            out_hbm.at[i, pl.ds(j * TT - 1, TT), 0, :],
            sem.at[slot]).start()

    @pl.when(lin == total - 1)
    def _():
        _wait_for(lin - 1)
        _wait_for(lin)


@functools.partial(jax.jit, static_argnames=())
def kernel(spikes, weight, bias):
    b, t, n, _ = spikes.shape
    nt = t // TT
    ttb = TT // 8
    x = jnp.reshape(spikes, (b, t, n))
    w = jnp.transpose(weight, (2, 0, 1))
    bias2 = bias[None, :]
    out = pl.pallas_call(
        _conv_body,
        grid=(b, nt),
        in_specs=[
            pl.BlockSpec((1, TT, n), lambda i, j: (i, j, 0)),
            pl.BlockSpec((1, 8, n),
                         lambda i, j: (i, jnp.maximum(j * ttb - 1, 0), 0)),
            pl.BlockSpec((K, n, n), lambda i, j: (0, 0, 0)),
            pl.BlockSpec((1, n), lambda i, j: (0, 0)),
        ],
        out_specs=pl.BlockSpec(memory_space=pl.ANY),
        out_shape=jax.ShapeDtypeStruct((b, t - 1, 1, n), jnp.float32),
        scratch_shapes=[
            pltpu.MemorySpace.VMEM((2, TT, n), jnp.float32),
            pltpu.SemaphoreType.DMA((2,)),
        ],
    )(x, x, w, bias2)
    return jnp.reshape(out, (b, t - 1, n, 1))


# speedup vs baseline: 1.8061x; 1.0671x over previous
import functools

import jax
import jax.numpy as jnp
from jax.experimental import pallas as pl
from jax.experimental.pallas import tpu as pltpu

NUM_VARS = 128
K = 5
TT = 8192


def _conv_body(xc_ref, xh_ref, w_ref, b_ref, out_hbm, xs_ref, sem):
    i = pl.program_id(0)
    j = pl.program_id(1)
    nt = pl.num_programs(1)
    lin = i * nt + j
    total = pl.num_programs(0) * nt
    slot = jax.lax.rem(lin, 2)

    def _wait_for(lin2):
        i2 = jax.lax.div(lin2, nt)
        j2 = jax.lax.rem(lin2, nt)
        slot2 = jax.lax.rem(lin2, 2)

        @pl.when(j2 == 0)
        def _():
            pltpu.make_async_copy(
                xs_ref.at[slot2, 1:TT],
                out_hbm.at[i2, :TT - 1, 0, :],
                sem.at[slot2]).wait()

        @pl.when(j2 != 0)
        def _():
            pltpu.make_async_copy(
                xs_ref.at[slot2, 0:TT],
                out_hbm.at[i2, pl.ds(j2 * TT - 1, TT), 0, :],
                sem.at[slot2]).wait()

    @pl.when(lin >= 2)
    def _():
        _wait_for(lin - 2)

    halo8 = jnp.where(j == 0, 0.0, xh_ref[0]).astype(jnp.bfloat16)
    xcb = xc_ref[0].astype(jnp.bfloat16)

    s0 = jnp.concatenate([halo8[4:8], xcb[:TT - 4]], axis=0)
    s2 = jnp.concatenate([halo8[6:8], xcb[:TT - 2]], axis=0)

    def dot(a, wk):
        return jax.lax.dot_general(
            a, wk, dimension_numbers=(((1,), (1,)), ((), ())),
            preferred_element_type=jnp.float32)

    w0, w1, w2, w3, w4 = (w_ref[k].astype(jnp.bfloat16) for k in range(K))
    row = jax.lax.broadcasted_iota(jnp.int32, (NUM_VARS, NUM_VARS), 0)
    col = jax.lax.broadcasted_iota(jnp.int32, (NUM_VARS, NUM_VARS), 1)
    w4 = jnp.where(row == col, 0.0, w4)

    odd = dot(s2, w1) + dot(xcb, w3)
    hrow = dot(halo8, w1)[5:6] + dot(halo8, w3)[7:8]
    odd_sh = jnp.concatenate([hrow, odd[:TT - 1]], axis=0)

    acc = (jnp.broadcast_to(b_ref[0][None, :], (TT, NUM_VARS)).astype(jnp.float32)
           + dot(s0, w0) + dot(s2, w2) + dot(xcb, w4) + odd_sh)
    xs_ref[slot] = acc

    @pl.when(j == 0)
    def _():
        pltpu.make_async_copy(
            xs_ref.at[slot, 1:TT],
            out_hbm.at[i, :TT - 1, 0, :],
            sem.at[slot]).start()

    @pl.when(j != 0)
    def _():
        pltpu.make_async_copy(
            xs_ref.at[slot, 0:TT],
            out_hbm.at[i, pl.ds(j * TT - 1, TT), 0, :],
            sem.at[slot]).start()

    @pl.when(lin == total - 1)
    def _():
        _wait_for(lin - 1)
        _wait_for(lin)


@functools.partial(jax.jit, static_argnames=())
def kernel(spikes, weight, bias):
    b, t, n, _ = spikes.shape
    nt = t // TT
    ttb = TT // 8
    x = jnp.reshape(spikes, (b, t, n))
    w = jnp.transpose(weight, (2, 0, 1))
    bias2 = bias[None, :]
    out = pl.pallas_call(
        _conv_body,
        grid=(b, nt),
        in_specs=[
            pl.BlockSpec((1, TT, n), lambda i, j: (i, j, 0)),
            pl.BlockSpec((1, 8, n),
                         lambda i, j: (i, jnp.maximum(j * ttb - 1, 0), 0)),
            pl.BlockSpec((K, n, n), lambda i, j: (0, 0, 0)),
            pl.BlockSpec((1, n), lambda i, j: (0, 0)),
        ],
        out_specs=pl.BlockSpec(memory_space=pl.ANY),
        out_shape=jax.ShapeDtypeStruct((b, t - 1, 1, n), jnp.float32),
        scratch_shapes=[
            pltpu.MemorySpace.VMEM((2, TT, n), jnp.float32),
            pltpu.SemaphoreType.DMA((2,)),
        ],
    )(x, x, w, bias2)
    return jnp.reshape(out, (b, t - 1, n, 1))


# speedup vs baseline: 1.8274x; 1.0118x over previous
import functools

import jax
import jax.numpy as jnp
from jax.experimental import pallas as pl
from jax.experimental.pallas import tpu as pltpu

NUM_VARS = 128
K = 5


def _conv_body(xc_ref, w_ref, b_ref, out_hbm, xs_ref, sem):
    i = pl.program_id(0)
    nb = pl.num_programs(0)
    tt = xc_ref.shape[1]
    slot = jax.lax.rem(i, 2)
    n = NUM_VARS

    def _wait_for(i2):
        slot2 = jax.lax.rem(i2, 2)
        pltpu.make_async_copy(
            xs_ref.at[slot2, 1:tt],
            out_hbm.at[i2, :tt - 1, 0, :],
            sem.at[slot2]).wait()

    @pl.when(i >= 2)
    def _():
        _wait_for(i - 2)

    xcb = xc_ref[0].astype(jnp.bfloat16)
    zb = jnp.zeros((4, n), jnp.bfloat16)
    s0 = jnp.concatenate([zb, xcb[:tt - 4]], axis=0)
    s2 = jnp.concatenate([zb[:2], xcb[:tt - 2]], axis=0)

    def dot(a, wk):
        return jax.lax.dot_general(
            a, wk, dimension_numbers=(((1,), (1,)), ((), ())),
            preferred_element_type=jnp.float32)

    w0, w1, w2, w3, w4 = (w_ref[k].astype(jnp.bfloat16) for k in range(K))
    row = jax.lax.broadcasted_iota(jnp.int32, (n, n), 0)
    col = jax.lax.broadcasted_iota(jnp.int32, (n, n), 1)
    w4 = jnp.where(row == col, 0.0, w4)

    odd = dot(s2, w1) + dot(xcb, w3)
    odd_sh = jnp.concatenate([jnp.zeros((1, n), jnp.float32),
                              odd[:tt - 1]], axis=0)

    acc = (jnp.broadcast_to(b_ref[0][None, :], (tt, n)).astype(jnp.float32)
           + dot(s0, w0) + dot(s2, w2) + dot(xcb, w4) + odd_sh)
    xs_ref[slot] = acc

    pltpu.make_async_copy(
        xs_ref.at[slot, 1:tt],
        out_hbm.at[i, :tt - 1, 0, :],
        sem.at[slot]).start()

    @pl.when(i == nb - 1)
    def _():
        @pl.when(nb >= 2)
        def _():
            _wait_for(i - 1)
        _wait_for(i)


@functools.partial(jax.jit, static_argnames=())
def kernel(spikes, weight, bias):
    b, t, n, _ = spikes.shape
    x = jnp.reshape(spikes, (b, t, n))
    w = jnp.transpose(weight, (2, 0, 1))
    bias2 = bias[None, :]
    out = pl.pallas_call(
        _conv_body,
        grid=(b,),
        in_specs=[
            pl.BlockSpec((1, t, n), lambda i: (i, 0, 0)),
            pl.BlockSpec((K, n, n), lambda i: (0, 0, 0)),
            pl.BlockSpec((1, n), lambda i: (0, 0)),
        ],
        out_specs=pl.BlockSpec(memory_space=pl.ANY),
        out_shape=jax.ShapeDtypeStruct((b, t - 1, 1, n), jnp.float32),
        scratch_shapes=[
            pltpu.MemorySpace.VMEM((2, t, n), jnp.float32),
            pltpu.SemaphoreType.DMA((2,)),
        ],
    )(x, w, bias2)
    return jnp.reshape(out, (b, t - 1, n, 1))


# speedup vs baseline: 1.9245x; 1.0532x over previous
import functools

import jax
import jax.numpy as jnp
from jax.experimental import pallas as pl
from jax.experimental.pallas import tpu as pltpu

NUM_VARS = 128
K = 5
NSLOTS = 3
NH = 2


def _conv_body(xc_ref, w_ref, b_ref, out_hbm, xs_ref, sem):
    i = pl.program_id(0)
    nb = pl.num_programs(0)
    tt = xc_ref.shape[1]
    ht = tt // NH
    slot = jax.lax.rem(i, NSLOTS)
    n = NUM_VARS

    def _wait_for(i2, h):
        slot2 = jax.lax.rem(i2, NSLOTS)
        if h == 0:
            pltpu.make_async_copy(
                xs_ref.at[slot2, 1:ht],
                out_hbm.at[i2, :ht - 1, 0, :],
                sem.at[slot2, 0]).wait()
        else:
            pltpu.make_async_copy(
                xs_ref.at[slot2, ht:tt],
                out_hbm.at[i2, ht - 1:tt - 1, 0, :],
                sem.at[slot2, 1]).wait()

    @pl.when(i >= NSLOTS)
    def _():
        _wait_for(i - NSLOTS, 0)
        _wait_for(i - NSLOTS, 1)

    def dot(a, wk):
        return jax.lax.dot_general(
            a, wk, dimension_numbers=(((1,), (1,)), ((), ())),
            preferred_element_type=jnp.float32)

    w0, w1, w2, w3, w4 = (w_ref[k].astype(jnp.bfloat16) for k in range(K))
    row = jax.lax.broadcasted_iota(jnp.int32, (n, n), 0)
    col = jax.lax.broadcasted_iota(jnp.int32, (n, n), 1)
    w4 = jnp.where(row == col, 0.0, w4)
    bias = jnp.broadcast_to(b_ref[0][None, :], (ht, n)).astype(jnp.float32)

    for h in range(NH):
        base = h * ht
        xch = xc_ref[0, base:base + ht].astype(jnp.bfloat16)
        if h == 0:
            halo8 = jnp.zeros((8, n), jnp.bfloat16)
            hrow = jnp.zeros((1, n), jnp.float32)
        else:
            halo8 = xc_ref[0, base - 8:base].astype(jnp.bfloat16)
            hrow = dot(halo8, w1)[5:6] + dot(halo8, w3)[7:8]
        s0 = jnp.concatenate([halo8[4:8], xch[:ht - 4]], axis=0)
        s2 = jnp.concatenate([halo8[6:8], xch[:ht - 2]], axis=0)

        odd = dot(s2, w1) + dot(xch, w3)
        odd_sh = jnp.concatenate([hrow, odd[:ht - 1]], axis=0)
        acc = bias + dot(s0, w0) + dot(s2, w2) + dot(xch, w4) + odd_sh
        xs_ref[slot, base:base + ht] = acc

        if h == 0:
            pltpu.make_async_copy(
                xs_ref.at[slot, 1:ht],
                out_hbm.at[i, :ht - 1, 0, :],
                sem.at[slot, 0]).start()
        else:
            pltpu.make_async_copy(
                xs_ref.at[slot, ht:tt],
                out_hbm.at[i, ht - 1:tt - 1, 0, :],
                sem.at[slot, 1]).start()

    @pl.when(i == nb - 1)
    def _():
        for d in range(NSLOTS - 1, -1, -1):
            @pl.when(nb >= d + 1)
            def _():
                _wait_for(i - d, 0)
                _wait_for(i - d, 1)


@functools.partial(jax.jit, static_argnames=())
def kernel(spikes, weight, bias):
    b, t, n, _ = spikes.shape
    x = jnp.reshape(spikes, (b, t, n))
    w = jnp.transpose(weight, (2, 0, 1))
    bias2 = bias[None, :]
    out = pl.pallas_call(
        _conv_body,
        grid=(b,),
        in_specs=[
            pl.BlockSpec((1, t, n), lambda i: (i, 0, 0)),
            pl.BlockSpec((K, n, n), lambda i: (0, 0, 0)),
            pl.BlockSpec((1, n), lambda i: (0, 0)),
        ],
        out_specs=pl.BlockSpec(memory_space=pl.ANY),
        out_shape=jax.ShapeDtypeStruct((b, t - 1, 1, n), jnp.float32),
        scratch_shapes=[
            pltpu.MemorySpace.VMEM((NSLOTS, t, n), jnp.float32),
            pltpu.SemaphoreType.DMA((NSLOTS, 2)),
        ],
    )(x, w, bias2)
    return jnp.reshape(out, (b, t - 1, n, 1))
